# Initial kernel scaffold; baseline (speedup 1.0000x reference)
import functools

import jax
import jax.numpy as jnp
from jax import lax
from jax.experimental import pallas as pl
from jax.experimental.pallas import tpu as pltpu
from jax.experimental.pallas import tpu_sc as plsc

N = 10000
D = 128
E = 320000

NC = 2
NS = 16
NW = NC * NS
K = 128
CHUNKS = -(-E // (NW * K))
T = K * CHUNKS
E_PAD = NW * T
N_ACC = 10240
ZROWS = N_ACC // NS
OROWS = N // NS

_mesh = plsc.VectorSubcoreMesh(core_axis_name="c", subcore_axis_name="s")


@functools.partial(
    pl.kernel,
    out_type=(
        jax.ShapeDtypeStruct((NC, N, D), jnp.float32),
        jax.ShapeDtypeStruct((NC, N, 16), jnp.float32),
    ),
    mesh=_mesh,
    scratch_types=[
        pltpu.VMEM_SHARED((N_ACC, D), jnp.float32),
        pltpu.VMEM_SHARED((N_ACC, 16), jnp.float32),
        pltpu.VMEM((K,), jnp.int32),
        pltpu.VMEM((K,), jnp.int32),
        pltpu.VMEM((K, D), jnp.float32),
        pltpu.VMEM((K, 16), jnp.float32),
        pltpu.SemaphoreType.DMA,
    ],
)
def _agg_cnt(x_hbm, srcp, dstp, z128, z16, ones16, out_p, out_c,
             acc, cnt_acc, src_v, dst_v, rows_v, ones_v, sem):
    c = lax.axis_index("c")
    s = lax.axis_index("s")
    pltpu.sync_copy(z128, acc.at[pl.ds(s * ZROWS, ZROWS)])
    pltpu.sync_copy(z16, cnt_acc.at[pl.ds(s * ZROWS, ZROWS)])
    pltpu.sync_copy(ones16, ones_v)
    plsc.subcore_barrier()
    base = (c * NS + s) * T

    def chunk(i, carry):
        off = base + i * K
        pltpu.sync_copy(srcp.at[pl.ds(off, K)], src_v)
        pltpu.sync_copy(dstp.at[pl.ds(off, K)], dst_v)
        pltpu.async_copy(x_hbm.at[src_v], rows_v, sem).wait()
        pltpu.sync_copy(rows_v, acc.at[dst_v], add=True)
        pltpu.sync_copy(ones_v, cnt_acc.at[dst_v], add=True)
        return carry

    lax.fori_loop(0, CHUNKS, chunk, 0)
    plsc.subcore_barrier()
    orow = s * OROWS
    pltpu.sync_copy(acc.at[pl.ds(orow, OROWS)], out_p.at[c, pl.ds(orow, OROWS)])
    pltpu.sync_copy(cnt_acc.at[pl.ds(orow, OROWS)], out_c.at[c, pl.ds(orow, OROWS)])


@functools.partial(
    pl.kernel,
    out_type=jax.ShapeDtypeStruct((NC, N, D), jnp.float32),
    mesh=_mesh,
    scratch_types=[
        pltpu.VMEM_SHARED((N_ACC, D), jnp.float32),
        pltpu.VMEM((K,), jnp.int32),
        pltpu.VMEM((K,), jnp.int32),
        pltpu.VMEM((K, D), jnp.float32),
        pltpu.SemaphoreType.DMA,
    ],
)
def _agg(x_hbm, srcp, dstp, z128, out_p, acc, src_v, dst_v, rows_v, sem):
    c = lax.axis_index("c")
    s = lax.axis_index("s")
    pltpu.sync_copy(z128, acc.at[pl.ds(s * ZROWS, ZROWS)])
    plsc.subcore_barrier()
    base = (c * NS + s) * T

    def chunk(i, carry):
        off = base + i * K
        pltpu.sync_copy(srcp.at[pl.ds(off, K)], src_v)
        pltpu.sync_copy(dstp.at[pl.ds(off, K)], dst_v)
        pltpu.async_copy(x_hbm.at[src_v], rows_v, sem).wait()
        pltpu.sync_copy(rows_v, acc.at[dst_v], add=True)
        return carry

    lax.fori_loop(0, CHUNKS, chunk, 0)
    plsc.subcore_barrier()
    orow = s * OROWS
    pltpu.sync_copy(acc.at[pl.ds(orow, OROWS)], out_p.at[c, pl.ds(orow, OROWS)])


RB = 1000
G = N // RB


def _combine_body(p_ref, c_ref, x_ref, wl_ref, wr_ref, b_ref, o_ref, *, relu):
    cnt = c_ref[0, :, 0:1] + c_ref[1, :, 0:1]
    inv = 1.0 / jnp.maximum(cnt, 1.0)
    mean = (p_ref[0] + p_ref[1]) * inv
    dn = (((1,), (1,)), ((), ()))
    acc = lax.dot_general(mean, wl_ref[...], dn, preferred_element_type=jnp.float32)
    acc = acc + lax.dot_general(x_ref[...], wr_ref[...], dn,
                                preferred_element_type=jnp.float32)
    acc = acc + b_ref[...]
    if relu:
        acc = jnp.maximum(acc, 0.0)
    o_ref[...] = acc


def _combine(p, cnt2, x, Wl, Wr, b, relu):
    return pl.pallas_call(
        functools.partial(_combine_body, relu=relu),
        grid=(G,),
        in_specs=[
            pl.BlockSpec((2, RB, D), lambda i: (0, i, 0)),
            pl.BlockSpec((2, RB, 16), lambda i: (0, i, 0)),
            pl.BlockSpec((RB, D), lambda i: (i, 0)),
            pl.BlockSpec((D, D), lambda i: (0, 0)),
            pl.BlockSpec((D, D), lambda i: (0, 0)),
            pl.BlockSpec((1, D), lambda i: (0, 0)),
        ],
        out_specs=pl.BlockSpec((RB, D), lambda i: (i, 0)),
        out_shape=jax.ShapeDtypeStruct((N, D), jnp.float32),
    )(p, cnt2, x, Wl, Wr, b.reshape(1, D))


def kernel(x, edge_index, W1l, W1r, b1, W2l, W2r, b2, W3l, W3r, b3):
    src = edge_index[0]
    dst = edge_index[1]
    pad = E_PAD - E
    srcp = jnp.concatenate([src, jnp.zeros((pad,), jnp.int32)])
    dstp = jnp.concatenate([dst, jnp.full((pad,), N, jnp.int32)])
    z128 = jnp.zeros((ZROWS, D), jnp.float32)
    z16 = jnp.zeros((ZROWS, 16), jnp.float32)
    ones16 = jnp.ones((K, 16), jnp.float32)

    p1, cnt2 = _agg_cnt(x, srcp, dstp, z128, z16, ones16)
    h1 = _combine(p1, cnt2, x, W1l, W1r, b1, relu=True)
    p2 = _agg(h1, srcp, dstp, z128)
    h2 = _combine(p2, cnt2, h1, W2l, W2r, b2, relu=True)
    p3 = _agg(h2, srcp, dstp, z128)
    return _combine(p3, cnt2, h2, W3l, W3r, b3, relu=False)


# speedup vs baseline: 3.8654x; 3.8654x over previous
import functools

import jax
import jax.numpy as jnp
from jax import lax
from jax.experimental import pallas as pl
from jax.experimental.pallas import tpu as pltpu
from jax.experimental.pallas import tpu_sc as plsc

N = 10000
D = 128
E = 320000

NC = 2
NS = 16
NW = NC * NS
K = 128
CHUNKS = -(-E // (NW * K))
T = K * CHUNKS
E_PAD = NW * T
N_ACC = 10240
ZROWS = N_ACC // NS


@functools.cache
def _sc_aggs():
    mesh = plsc.VectorSubcoreMesh(core_axis_name="c", subcore_axis_name="s",
                                  num_cores=NC, num_subcores=NS)

    @functools.partial(
        pl.kernel,
        out_type=jax.ShapeDtypeStruct((NC, N_ACC, D), jnp.float32),
        mesh=mesh,
        scratch_types=[
            pltpu.VMEM_SHARED((N_ACC, D), jnp.float32),
            pltpu.VMEM((K,), jnp.int32),
            pltpu.VMEM((K,), jnp.int32),
            pltpu.VMEM((K, D), jnp.float32),
            pltpu.SemaphoreType.DMA,
        ],
    )
    def _agg(x_hbm, srcp, dstp, z128, out_p, acc, src_v, dst_v, rows_v, sem):
        c = lax.axis_index("c")
        s = lax.axis_index("s")
        pltpu.sync_copy(z128, acc.at[pl.ds(s * ZROWS, ZROWS)])
        plsc.subcore_barrier()
        base = (c * NS + s) * T

        def chunk(i, carry):
            off = base + i * K
            pltpu.sync_copy(srcp.at[pl.ds(off, K)], src_v)
            pltpu.sync_copy(dstp.at[pl.ds(off, K)], dst_v)
            pltpu.async_copy(x_hbm.at[src_v], rows_v, sem).wait()
            pltpu.sync_copy(rows_v, acc.at[dst_v], add=True)
            return carry

        lax.fori_loop(0, CHUNKS, chunk, 0)
        plsc.subcore_barrier()
        orow = s * ZROWS
        pltpu.sync_copy(acc.at[pl.ds(orow, ZROWS)], out_p.at[c, pl.ds(orow, ZROWS)])

    @functools.partial(
        pl.kernel,
        out_type=jax.ShapeDtypeStruct((NC, N_ACC, D), jnp.float32),
        mesh=mesh,
        scratch_types=[
            pltpu.VMEM_SHARED((N_ACC, D), jnp.float32),
            pltpu.VMEM((K,), jnp.int32),
            pltpu.VMEM((K, D), jnp.float32),
        ],
    )
    def _cnt(dstp, z128, ones128, out_c, acc, dst_v, ones_v):
        c = lax.axis_index("c")
        s = lax.axis_index("s")
        pltpu.sync_copy(z128, acc.at[pl.ds(s * ZROWS, ZROWS)])
        pltpu.sync_copy(ones128, ones_v)
        plsc.subcore_barrier()
        base = (c * NS + s) * T

        def chunk(i, carry):
            off = base + i * K
            pltpu.sync_copy(dstp.at[pl.ds(off, K)], dst_v)
            pltpu.sync_copy(ones_v, acc.at[dst_v], add=True)
            return carry

        lax.fori_loop(0, CHUNKS, chunk, 0)
        plsc.subcore_barrier()
        orow = s * ZROWS
        pltpu.sync_copy(acc.at[pl.ds(orow, ZROWS)], out_c.at[c, pl.ds(orow, ZROWS)])

    return _agg, _cnt


RB = 1000
G = N // RB


def _combine_body(p_ref, c_ref, x_ref, wl_ref, wr_ref, b_ref, o_ref, *, relu):
    cnt = c_ref[0, :, 0:1] + c_ref[1, :, 0:1]
    inv = 1.0 / jnp.maximum(cnt, 1.0)
    mean = (p_ref[0] + p_ref[1]) * inv
    dn = (((1,), (1,)), ((), ()))
    acc = lax.dot_general(mean, wl_ref[...], dn, preferred_element_type=jnp.float32)
    acc = acc + lax.dot_general(x_ref[...], wr_ref[...], dn,
                                preferred_element_type=jnp.float32)
    acc = acc + b_ref[...]
    if relu:
        acc = jnp.maximum(acc, 0.0)
    o_ref[...] = acc


def _combine(p, cnt2, x, Wl, Wr, b, relu):
    return pl.pallas_call(
        functools.partial(_combine_body, relu=relu),
        grid=(G,),
        in_specs=[
            pl.BlockSpec((2, RB, D), lambda i: (0, i, 0)),
            pl.BlockSpec((2, RB, D), lambda i: (0, i, 0)),
            pl.BlockSpec((RB, D), lambda i: (i, 0)),
            pl.BlockSpec((D, D), lambda i: (0, 0)),
            pl.BlockSpec((D, D), lambda i: (0, 0)),
            pl.BlockSpec((1, D), lambda i: (0, 0)),
        ],
        out_specs=pl.BlockSpec((RB, D), lambda i: (i, 0)),
        out_shape=jax.ShapeDtypeStruct((N, D), jnp.float32),
    )(p, cnt2, x, Wl, Wr, b.reshape(1, D))


def kernel(x, edge_index, W1l, W1r, b1, W2l, W2r, b2, W3l, W3r, b3):
    src = edge_index[0]
    dst = edge_index[1]
    pad = E_PAD - E
    srcp = jnp.concatenate([src, jnp.zeros((pad,), jnp.int32)])
    dstp = jnp.concatenate([dst, jnp.full((pad,), N, jnp.int32)])
    z128 = jnp.zeros((ZROWS, D), jnp.float32)
    ones128 = jnp.ones((K, D), jnp.float32)

    agg_fn, cnt_fn = _sc_aggs()
    cnt2 = cnt_fn(dstp, z128, ones128)
    p1 = agg_fn(x, srcp, dstp, z128)
    h1 = _combine(p1, cnt2, x, W1l, W1r, b1, relu=True)
    p2 = agg_fn(h1, srcp, dstp, z128)
    h2 = _combine(p2, cnt2, h1, W2l, W2r, b2, relu=True)
    p3 = agg_fn(h2, srcp, dstp, z128)
    return _combine(p3, cnt2, h2, W3l, W3r, b3, relu=False)
